# Initial kernel scaffold; baseline (speedup 1.0000x reference)
import jax
import jax.numpy as jnp
from jax.experimental import pallas as pl


def kernel(x, edge_index, W, b):
    raise NotImplementedError("write your pallas kernel here")


# speedup vs baseline: 19.5287x; 19.5287x over previous
import functools

import jax
import jax.numpy as jnp
from jax import lax
from jax.experimental import pallas as pl
from jax.experimental.pallas import tpu as pltpu
from jax.experimental.pallas import tpu_sc as plsc

NC = 2
NS = 16
NW = NC * NS
CHUNK = 128
HISTW = 16


def _mesh():
    return plsc.VectorSubcoreMesh(core_axis_name="c", subcore_axis_name="s")


_SC_PARAMS = pltpu.CompilerParams(use_tc_tiling_on_sc=False)


def _hist_kernel(ndp, rpt, per_tile, dtype=jnp.float32):
    n_chunks = per_tile // CHUNK

    @functools.partial(
        pl.kernel,
        out_type=jax.ShapeDtypeStruct((NC, ndp, HISTW), dtype),
        mesh=_mesh(),
        compiler_params=_SC_PARAMS,
        scratch_types=[
            pltpu.VMEM((1, CHUNK), jnp.int32),
            pltpu.VMEM((CHUNK, HISTW), dtype),
            pltpu.VMEM((rpt, HISTW), dtype),
            pltpu.VMEM_SHARED((ndp, HISTW), dtype),
        ],
    )
    def hist(dst_hbm, ones_hbm, zeros_hbm, out_hbm, idx_v, ones_v, zb, deg_sh):
        c = lax.axis_index("c")
        s = lax.axis_index("s")
        wid = s * NC + c
        r0 = s * rpt
        pltpu.sync_copy(zeros_hbm.at[pl.ds(r0, rpt)], zb)
        pltpu.sync_copy(zb, deg_sh.at[pl.ds(r0, rpt)])
        pltpu.sync_copy(ones_hbm, ones_v)
        plsc.subcore_barrier()
        ebase = wid * per_tile

        def body(i, carry):
            pltpu.sync_copy(dst_hbm.at[pl.ds(ebase + i * CHUNK, CHUNK)],
                            idx_v.at[0])
            pltpu.sync_copy(ones_v, deg_sh.at[idx_v.at[0]], add=True)
            return carry

        lax.fori_loop(0, n_chunks, body, 0)
        plsc.subcore_barrier()
        pltpu.sync_copy(deg_sh.at[pl.ds(r0, rpt)],
                        out_hbm.at[c, pl.ds(r0, rpt)])

    return hist


def _edges_kernel(n_nodes, ndp, rpt, per_tile, h):
    n_chunks = per_tile // CHUNK

    @functools.partial(
        pl.kernel,
        out_type=jax.ShapeDtypeStruct((NC, ndp, h), jnp.float32),
        mesh=_mesh(),
        compiler_params=_SC_PARAMS,
        scratch_types=[
            pltpu.VMEM((1, CHUNK), jnp.int32),
            pltpu.VMEM((1, CHUNK), jnp.int32),
            pltpu.VMEM((CHUNK, h), jnp.float32),
            pltpu.VMEM((rpt, h), jnp.float32),
            pltpu.VMEM_SHARED((ndp, h), jnp.float32),
        ],
    )
    def edges(xs_hbm, src_hbm, dst_hbm, zeros_hbm, out_hbm,
              sidx, didx, rows_v, zb, acc_sh):
        c = lax.axis_index("c")
        s = lax.axis_index("s")
        wid = s * NC + c
        r0 = s * rpt
        pltpu.sync_copy(zeros_hbm.at[pl.ds(r0, rpt)], zb)
        pltpu.sync_copy(zb, acc_sh.at[pl.ds(r0, rpt)])
        plsc.subcore_barrier()
        ebase = wid * per_tile

        def body(i, carry):
            off = ebase + i * CHUNK
            pltpu.sync_copy(src_hbm.at[pl.ds(off, CHUNK)], sidx.at[0])
            pltpu.sync_copy(dst_hbm.at[pl.ds(off, CHUNK)], didx.at[0])
            pltpu.sync_copy(xs_hbm.at[sidx.at[0]], rows_v)
            pltpu.sync_copy(rows_v, acc_sh.at[didx.at[0]], add=True)
            return carry

        lax.fori_loop(0, n_chunks, body, 0)
        plsc.subcore_barrier()
        pltpu.sync_copy(acc_sh.at[pl.ds(r0, rpt)],
                        out_hbm.at[c, pl.ds(r0, rpt)])

    return edges


def _pick_block(n):
    for cand in (1024, 1000, 800, 512, 400, 256, 200, 128, 104, 80, 64, 40,
                 32, 16, 8):
        if n % cand == 0:
            return cand
    return n


def _dense_body(x_ref, w_ref, dp_ref, xs_ref):
    deg = dp_ref[0, :, 0:1] + dp_ref[1, :, 0:1] + 1.0
    dinv = lax.rsqrt(deg)
    xw = jnp.dot(x_ref[...], w_ref[...], preferred_element_type=jnp.float32)
    xs_ref[...] = xw * dinv


def _epi_body(acc_ref, xs_ref, dp_ref, b_ref, o_ref):
    deg = dp_ref[0, :, 0:1] + dp_ref[1, :, 0:1] + 1.0
    dinv = lax.rsqrt(deg)
    tot = acc_ref[0] + acc_ref[1] + xs_ref[...]
    o_ref[...] = jnp.maximum(tot * dinv + b_ref[...], 0.0)


def kernel(x, edge_index, W, b):
    n, in_dim = x.shape
    h = W.shape[1]
    e = edge_index.shape[1]
    src = edge_index[0].astype(jnp.int32)
    dst = edge_index[1].astype(jnp.int32)

    per_tile = -(-e // (NW * CHUNK)) * CHUNK
    ep = NW * per_tile
    pad = ep - e
    if pad:
        src = jnp.concatenate([src, jnp.zeros((pad,), jnp.int32)])
        dst = jnp.concatenate([dst, jnp.full((pad,), n, jnp.int32)])
    ndp = -(-(n + 1) // (NS * 8)) * (NS * 8)
    rpt = ndp // NS

    ones16 = jnp.ones((CHUNK, HISTW), jnp.float32)
    zeros16 = jnp.zeros((ndp, HISTW), jnp.float32)
    zeros_h = jnp.zeros((ndp, h), jnp.float32)

    degp = _hist_kernel(ndp, rpt, per_tile)(dst, ones16, zeros16)

    blk = _pick_block(n)
    grid = (n // blk,)
    xs = pl.pallas_call(
        _dense_body,
        grid=grid,
        in_specs=[
            pl.BlockSpec((blk, in_dim), lambda i: (i, 0)),
            pl.BlockSpec((in_dim, h), lambda i: (0, 0)),
            pl.BlockSpec((NC, blk, HISTW), lambda i: (0, i, 0)),
        ],
        out_specs=pl.BlockSpec((blk, h), lambda i: (i, 0)),
        out_shape=jax.ShapeDtypeStruct((n, h), jnp.float32),
    )(x, W, degp)

    accp = _edges_kernel(n, ndp, rpt, per_tile, h)(xs, src, dst, zeros_h)

    out = pl.pallas_call(
        _epi_body,
        grid=grid,
        in_specs=[
            pl.BlockSpec((NC, blk, h), lambda i: (0, i, 0)),
            pl.BlockSpec((blk, h), lambda i: (i, 0)),
            pl.BlockSpec((NC, blk, HISTW), lambda i: (0, i, 0)),
            pl.BlockSpec((1, h), lambda i: (0, 0)),
        ],
        out_specs=pl.BlockSpec((blk, h), lambda i: (i, 0)),
        out_shape=jax.ShapeDtypeStruct((n, h), jnp.float32),
    )(accp, xs, degp, b.reshape(1, h))
    return out
